# baseline (speedup 1.0000x reference)
import jax
import jax.numpy as jnp
from jax import lax
from jax.experimental import pallas as pl
from jax.experimental.pallas import tpu as pltpu
from jax.experimental.pallas import tpu_sc as plsc

_VOCAB = 1000000
_EMBED = 16
_KCTX = 5
_BATCH = 16384
_NC = 2
_NS = 16
_NW = _NC * _NS
_BPW = _BATCH // _NW
_CHUNK = 128
_NCH_T = _BPW // _CHUNK
_NCH_C = _BPW * _KCTX // _CHUNK
_GROUPS = _BPW // 16


def _sc_body(tgt_idx_hbm, ctx_idx_hbm, ttab_hbm, ctab_hbm, out_hbm,
             tgt_idx_v, ctx_idx_v, tgt_rows, ctx_rows, out_v, sem):
    wid = lax.axis_index("s") * _NC + lax.axis_index("c")
    base = wid * _BPW

    pltpu.sync_copy(tgt_idx_hbm.at[pl.ds(base, _BPW)], tgt_idx_v)
    pltpu.sync_copy(ctx_idx_hbm.at[pl.ds(base * _KCTX, _BPW * _KCTX)],
                    ctx_idx_v)

    copies = []
    for j in range(_NCH_T):
        copies.append(pltpu.async_copy(
            ttab_hbm.at[tgt_idx_v.at[pl.ds(j * _CHUNK, _CHUNK)]],
            tgt_rows.at[pl.ds(j * _CHUNK, _CHUNK)], sem))
    for j in range(_NCH_C):
        copies.append(pltpu.async_copy(
            ctab_hbm.at[ctx_idx_v.at[pl.ds(j * _CHUNK, _CHUNK)]],
            ctx_rows.at[pl.ds(j * _CHUNK, _CHUNK)], sem))
    for c in copies:
        c.wait()

    def group(g, carry):
        row_ids = g * 16 + lax.iota(jnp.int32, 16)
        tcol = [
            plsc.load_gather(
                tgt_rows, [row_ids, jnp.full((16,), d, jnp.int32)])
            for d in range(_EMBED)
        ]
        crow_base = row_ids * _KCTX
        for n in range(_KCTX):
            crow = crow_base + n
            acc = tcol[0] * plsc.load_gather(
                ctx_rows, [crow, jnp.full((16,), 0, jnp.int32)])
            for d in range(1, _EMBED):
                acc = acc + tcol[d] * plsc.load_gather(
                    ctx_rows, [crow, jnp.full((16,), d, jnp.int32)])
            plsc.store_scatter(
                out_v, [row_ids, jnp.full((16,), n, jnp.int32)], acc)
        return carry

    lax.fori_loop(0, _GROUPS, group, 0)
    pltpu.sync_copy(out_v, out_hbm.at[pl.ds(base, _BPW), :])


@jax.jit
def _score(tgt_idx, ctx_idx, target_table, context_table):
    mesh = plsc.VectorSubcoreMesh(core_axis_name="c", subcore_axis_name="s")
    run = pl.kernel(
        _sc_body,
        mesh=mesh,
        compiler_params=pltpu.CompilerParams(
            needs_layout_passes=False, use_tc_tiling_on_sc=False),
        out_type=jax.ShapeDtypeStruct((_BATCH, _KCTX), jnp.float32),
        scratch_types=[
            pltpu.VMEM((_BPW,), jnp.int32),
            pltpu.VMEM((_BPW * _KCTX,), jnp.int32),
            pltpu.VMEM((_BPW, _EMBED), jnp.float32),
            pltpu.VMEM((_BPW * _KCTX, _EMBED), jnp.float32),
            pltpu.VMEM((_BPW, _KCTX), jnp.float32),
            pltpu.SemaphoreType.DMA,
        ],
    )
    return run(tgt_idx, ctx_idx, target_table, context_table)


def kernel(target, context, target_table, context_table):
    tgt_idx = target.reshape(_BATCH)
    ctx_idx = context.reshape(_BATCH * _KCTX)
    return _score(tgt_idx, ctx_idx, target_table, context_table)
